# baseline (speedup 1.0000x reference)
import functools

import jax
import jax.numpy as jnp
from jax import lax
from jax.experimental import pallas as pl
from jax.experimental.pallas import tpu as pltpu
from jax.experimental.pallas import tpu_sc as plsc

NS = 16
PAD = 32
CNT = 28

SC_CORES = 2
SC_SUBCORES = 16
NW = SC_CORES * SC_SUBCORES


def _node_body(x_ref, wtr_ref, a_ref, o_ref):
    xb = x_ref[...]
    wr = wtr_ref[..., 0] + wtr_ref[..., 1]
    a = a_ref[...]
    xs2 = jnp.dot(xb[:, :NS], wr, preferred_element_type=jnp.float32)
    y = (jnp.dot(xs2, a[:NS, :], preferred_element_type=jnp.float32)
         + jnp.dot(xb[:, NS:], a[NS:, :], preferred_element_type=jnp.float32))
    bn = y.shape[0]
    o_ref[...] = jnp.concatenate(
        [y, jnp.ones((bn, 1), jnp.float32), jnp.zeros((bn, PAD - CNT - 1), jnp.float32)],
        axis=1)


def _edge_body(ea_ref, es_ref, m1t_ref, b1_ref, m2t_ref, b2_ref, bsht_ref, o_ref):
    h = jnp.maximum(jnp.dot(m1t_ref[...], ea_ref[...],
                            preferred_element_type=jnp.float32) + b1_ref[...], 0.0)
    ew = jnp.dot(m2t_ref[...], h, preferred_element_type=jnp.float32) + b2_ref[...]
    shp = jnp.dot(bsht_ref[...], es_ref[...], preferred_element_type=jnp.float32)
    d = ew * shp
    be = d.shape[1]
    o_ref[...] = jnp.concatenate(
        [d, jnp.ones((1, be), jnp.float32), jnp.zeros((PAD - CNT - 1, be), jnp.float32)],
        axis=0)


def _combine_body(p_ref, o_ref):
    pb = p_ref[0] + p_ref[1]
    cnt = jnp.maximum(pb[:, CNT:CNT + 1], 1.0)
    o_ref[...] = pb[:, :CNT] / cnt


def _sc_scatter(n_nodes, n_edges):
    ew_per_tile = n_edges // NW
    K = 128
    nch = ew_per_tile // K
    tail = ew_per_tile - nch * K
    rows_per_sub = n_nodes // SC_SUBCORES
    ZR = 125
    nz = rows_per_sub // ZR

    mesh = plsc.VectorSubcoreMesh(core_axis_name="c", subcore_axis_name="s",
                                  num_cores=SC_CORES, num_subcores=SC_SUBCORES)

    scratch = [
        pltpu.VMEM((K,), jnp.int32),
        pltpu.VMEM((K,), jnp.int32),
        pltpu.VMEM((K, PAD), jnp.float32),
        pltpu.VMEM((PAD, K), jnp.float32),
        pltpu.VMEM((ZR, PAD), jnp.float32),
        pltpu.VMEM_SHARED((n_nodes, PAD), jnp.float32),
        pltpu.SemaphoreType.DMA,
    ]
    if tail:
        scratch += [
            pltpu.VMEM((tail,), jnp.int32),
            pltpu.VMEM((tail,), jnp.int32),
            pltpu.VMEM((tail, PAD), jnp.float32),
            pltpu.VMEM((PAD, tail), jnp.float32),
        ]

    @functools.partial(
        pl.kernel,
        out_type=jax.ShapeDtypeStruct((SC_CORES, n_nodes, PAD), jnp.float32),
        mesh=mesh,
        scratch_types=scratch,
        compiler_params=pltpu.CompilerParams(use_tc_tiling_on_sc=False,
                                             needs_layout_passes=False),
    )
    def run(src_hbm, dst_hbm, y_hbm, d_hbm, out_hbm, si, di, rows, dv, zbuf,
            acc, sem, *tails):
        c = lax.axis_index("c")
        s = lax.axis_index("s")
        wid = c * SC_SUBCORES + s
        zero16 = jnp.zeros((16,), jnp.float32)

        def zb(i, carry):
            zbuf[i, pl.ds(0, 16)] = zero16
            zbuf[i, pl.ds(16, 16)] = zero16
            return carry
        lax.fori_loop(0, ZR, zb, 0)

        def zc(k, carry):
            pltpu.sync_copy(zbuf, acc.at[pl.ds(s * rows_per_sub + k * ZR, ZR)])
            return carry
        lax.fori_loop(0, nz, zc, 0)
        plsc.subcore_barrier()

        base0 = wid * ew_per_tile

        iota_lo = lax.iota(jnp.int32, 16)
        iota_hi = iota_lo + 16

        def do_chunk(base, si_, di_, rows_, dv_, kk):
            pltpu.sync_copy(src_hbm.at[pl.ds(base, kk)], si_)
            pltpu.sync_copy(dst_hbm.at[pl.ds(base, kk)], di_)
            pltpu.sync_copy(d_hbm.at[:, pl.ds(base, kk)], dv_)
            pltpu.async_copy(y_hbm.at[si_], rows_, sem).wait()

            def mul(i, carry):
                icol = jnp.full((16,), i, jnp.int32)
                v0 = plsc.load_gather(dv_, [iota_lo, icol])
                v1 = plsc.load_gather(dv_, [iota_hi, icol])
                rows_[i, pl.ds(0, 16)] = rows_[i, pl.ds(0, 16)] * v0
                rows_[i, pl.ds(16, 16)] = rows_[i, pl.ds(16, 16)] * v1
                return carry
            lax.fori_loop(0, kk, mul, 0)
            pltpu.sync_copy(rows_, acc.at[di_], add=True)

        def chunk(ch, carry):
            do_chunk(base0 + ch * K, si, di, rows, dv, K)
            return carry
        lax.fori_loop(0, nch, chunk, 0)
        if tail:
            sit, dit, rowst, dvt = tails
            do_chunk(base0 + nch * K, sit, dit, rowst, dvt, tail)

        plsc.subcore_barrier()
        pltpu.sync_copy(acc.at[pl.ds(s * rows_per_sub, rows_per_sub)],
                        out_hbm.at[c, pl.ds(s * rows_per_sub, rows_per_sub)])

    return run


def kernel(x, edge_index, edge_attr, edge_sh, Wt_r, Wt_i, M1, b1, M2, b2, A, Bsh):
    n_nodes = x.shape[1]
    n_edges = edge_index.shape[1]
    src = edge_index[0]
    dst = edge_index[1]

    BN = 2000
    y_pad = pl.pallas_call(
        _node_body,
        grid=(n_nodes // BN,),
        in_specs=[
            pl.BlockSpec((BN, x.shape[2]), lambda i: (i, 0)),
            pl.BlockSpec(Wt_r.shape, lambda i: (0, 0, 0)),
            pl.BlockSpec(A.shape, lambda i: (0, 0)),
        ],
        out_specs=pl.BlockSpec((BN, PAD), lambda i: (i, 0)),
        out_shape=jax.ShapeDtypeStruct((n_nodes, PAD), jnp.float32),
    )(x[0], Wt_r, A)

    BE = 6400
    ea_t = edge_attr.T
    es_t = edge_sh.T
    d_pad_t = pl.pallas_call(
        _edge_body,
        grid=(n_edges // BE,),
        in_specs=[
            pl.BlockSpec((ea_t.shape[0], BE), lambda i: (0, i)),
            pl.BlockSpec((es_t.shape[0], BE), lambda i: (0, i)),
            pl.BlockSpec(M1.shape, lambda i: (0, 0)),
            pl.BlockSpec((b1.shape[0], 1), lambda i: (0, 0)),
            pl.BlockSpec((M2.shape[1], M2.shape[0]), lambda i: (0, 0)),
            pl.BlockSpec((b2.shape[0], 1), lambda i: (0, 0)),
            pl.BlockSpec((Bsh.shape[1], Bsh.shape[0]), lambda i: (0, 0)),
        ],
        out_specs=pl.BlockSpec((PAD, BE), lambda i: (0, i)),
        out_shape=jax.ShapeDtypeStruct((PAD, n_edges), jnp.float32),
    )(ea_t, es_t, M1.T, b1.reshape(-1, 1), M2.T, b2.reshape(-1, 1), Bsh.T)

    partials = _sc_scatter(n_nodes, n_edges)(src, dst, y_pad, d_pad_t)

    out = pl.pallas_call(
        _combine_body,
        grid=(n_nodes // BN,),
        in_specs=[pl.BlockSpec((SC_CORES, BN, PAD), lambda i: (0, i, 0))],
        out_specs=pl.BlockSpec((BN, CNT), lambda i: (i, 0)),
        out_shape=jax.ShapeDtypeStruct((n_nodes, CNT), jnp.float32),
    )(partials)

    return out[None]


# speedup vs baseline: 2.1939x; 2.1939x over previous
import functools

import jax
import jax.numpy as jnp
from jax import lax
from jax.experimental import pallas as pl
from jax.experimental.pallas import tpu as pltpu
from jax.experimental.pallas import tpu_sc as plsc

NS = 16
PAD = 32
CNT = 28

SC_CORES = 2
SC_SUBCORES = 16
NW = SC_CORES * SC_SUBCORES


def _node_body(x_ref, wtr_ref, a_ref, o_ref):
    xb = x_ref[...]
    wr = wtr_ref[..., 0] + wtr_ref[..., 1]
    a = a_ref[...]
    xs2 = jnp.dot(xb[:, :NS], wr, preferred_element_type=jnp.float32)
    y = (jnp.dot(xs2, a[:NS, :], preferred_element_type=jnp.float32)
         + jnp.dot(xb[:, NS:], a[NS:, :], preferred_element_type=jnp.float32))
    bn = y.shape[0]
    o_ref[...] = jnp.concatenate(
        [y, jnp.ones((bn, 1), jnp.float32), jnp.zeros((bn, PAD - CNT - 1), jnp.float32)],
        axis=1)


def _edge_body(ea_ref, es_ref, m1t_ref, b1_ref, m2t_ref, b2_ref, bsht_ref, o_ref):
    h = jnp.maximum(jnp.dot(m1t_ref[...], ea_ref[...],
                            preferred_element_type=jnp.float32) + b1_ref[...], 0.0)
    ew = jnp.dot(m2t_ref[...], h, preferred_element_type=jnp.float32) + b2_ref[...]
    shp = jnp.dot(bsht_ref[...], es_ref[...], preferred_element_type=jnp.float32)
    d = ew * shp
    be = d.shape[1]
    d32 = jnp.concatenate(
        [d, jnp.ones((1, be), jnp.float32), jnp.zeros((PAD - CNT - 1, be), jnp.float32)],
        axis=0)
    o_ref[...] = d32.reshape(PAD // 8, 8, be // 128, 128).swapaxes(1, 2)


def _combine_body(p_ref, o_ref):
    pb = p_ref[0] + p_ref[1]
    cnt = jnp.maximum(pb[:, CNT:CNT + 1], 1.0)
    o_ref[...] = pb[:, :CNT] / cnt


def _sc_scatter(n_nodes, n_edges):
    K = 128
    ncols = n_edges // K
    cols_base = ncols // NW
    cols_extra = ncols - cols_base * NW
    rows_per_sub = n_nodes // SC_SUBCORES
    ZR = 125
    nz = rows_per_sub // ZR

    mesh = plsc.VectorSubcoreMesh(core_axis_name="c", subcore_axis_name="s",
                                  num_cores=SC_CORES, num_subcores=SC_SUBCORES)

    scratch = [
        pltpu.VMEM((K,), jnp.int32),
        pltpu.VMEM((K,), jnp.int32),
        pltpu.VMEM((K, PAD), jnp.float32),
        pltpu.VMEM((PAD // 8, 8, K), jnp.float32),
        pltpu.VMEM((ZR, PAD), jnp.float32),
        pltpu.VMEM_SHARED((n_nodes, PAD), jnp.float32),
        pltpu.SemaphoreType.DMA,
    ]

    @functools.partial(
        pl.kernel,
        out_type=jax.ShapeDtypeStruct((SC_CORES, n_nodes, PAD), jnp.float32),
        mesh=mesh,
        scratch_types=scratch,
        compiler_params=pltpu.CompilerParams(use_tc_tiling_on_sc=False,
                                             needs_layout_passes=False),
    )
    def run(src_hbm, dst_hbm, y_hbm, d_hbm, out_hbm, si, di, rows, dv, zbuf,
            acc, sem):
        c = lax.axis_index("c")
        s = lax.axis_index("s")
        wid = c * SC_SUBCORES + s
        zero16 = jnp.zeros((16,), jnp.float32)

        def zb(i, carry):
            zbuf[i, pl.ds(0, 16)] = zero16
            zbuf[i, pl.ds(16, 16)] = zero16
            return carry
        lax.fori_loop(0, ZR, zb, 0)

        def zc(k, carry):
            pltpu.sync_copy(zbuf, acc.at[pl.ds(s * rows_per_sub + k * ZR, ZR)])
            return carry
        lax.fori_loop(0, nz, zc, 0)
        plsc.subcore_barrier()

        col0 = wid * cols_base + jnp.minimum(wid, cols_extra)
        my_cols = cols_base + jnp.where(wid < cols_extra, 1, 0)

        iota16 = lax.iota(jnp.int32, 16)
        tr_lo = iota16 // 8
        tr_hi = tr_lo + 2
        r_all = iota16 % 8

        def chunk(ch, carry):
            col = col0 + ch
            base = col * K
            pltpu.sync_copy(src_hbm.at[pl.ds(base, K)], si)
            pltpu.sync_copy(dst_hbm.at[pl.ds(base, K)], di)
            pltpu.sync_copy(d_hbm.at[:, col], dv)
            pltpu.async_copy(y_hbm.at[si], rows, sem).wait()

            def mul(i, carry2):
                icol = jnp.full((16,), i, jnp.int32)
                v0 = plsc.load_gather(dv, [tr_lo, r_all, icol])
                v1 = plsc.load_gather(dv, [tr_hi, r_all, icol])
                rows[i, pl.ds(0, 16)] = rows[i, pl.ds(0, 16)] * v0
                rows[i, pl.ds(16, 16)] = rows[i, pl.ds(16, 16)] * v1
                return carry2
            lax.fori_loop(0, K, mul, 0)
            pltpu.sync_copy(rows, acc.at[di], add=True)
            return carry
        lax.fori_loop(0, my_cols, chunk, 0)

        plsc.subcore_barrier()
        pltpu.sync_copy(acc.at[pl.ds(s * rows_per_sub, rows_per_sub)],
                        out_hbm.at[c, pl.ds(s * rows_per_sub, rows_per_sub)])

    return run


def kernel(x, edge_index, edge_attr, edge_sh, Wt_r, Wt_i, M1, b1, M2, b2, A, Bsh):
    n_nodes = x.shape[1]
    n_edges = edge_index.shape[1]
    src = edge_index[0]
    dst = edge_index[1]

    BN = 2000
    y_pad = pl.pallas_call(
        _node_body,
        grid=(n_nodes // BN,),
        in_specs=[
            pl.BlockSpec((BN, x.shape[2]), lambda i: (i, 0)),
            pl.BlockSpec(Wt_r.shape, lambda i: (0, 0, 0)),
            pl.BlockSpec(A.shape, lambda i: (0, 0)),
        ],
        out_specs=pl.BlockSpec((BN, PAD), lambda i: (i, 0)),
        out_shape=jax.ShapeDtypeStruct((n_nodes, PAD), jnp.float32),
    )(x[0], Wt_r, A)

    BE = 6400
    ea_t = edge_attr.T
    es_t = edge_sh.T
    d_pad_t = pl.pallas_call(
        _edge_body,
        grid=(n_edges // BE,),
        in_specs=[
            pl.BlockSpec((ea_t.shape[0], BE), lambda i: (0, i)),
            pl.BlockSpec((es_t.shape[0], BE), lambda i: (0, i)),
            pl.BlockSpec(M1.shape, lambda i: (0, 0)),
            pl.BlockSpec((b1.shape[0], 1), lambda i: (0, 0)),
            pl.BlockSpec((M2.shape[1], M2.shape[0]), lambda i: (0, 0)),
            pl.BlockSpec((b2.shape[0], 1), lambda i: (0, 0)),
            pl.BlockSpec((Bsh.shape[1], Bsh.shape[0]), lambda i: (0, 0)),
        ],
        out_specs=pl.BlockSpec((PAD // 8, BE // 128, 8, 128), lambda i: (0, i, 0, 0)),
        out_shape=jax.ShapeDtypeStruct((PAD // 8, n_edges // 128, 8, 128),
                                       jnp.float32),
    )(ea_t, es_t, M1.T, b1.reshape(-1, 1), M2.T, b2.reshape(-1, 1), Bsh.T)

    partials = _sc_scatter(n_nodes, n_edges)(src, dst, y_pad, d_pad_t)

    out = pl.pallas_call(
        _combine_body,
        grid=(n_nodes // BN,),
        in_specs=[pl.BlockSpec((SC_CORES, BN, PAD), lambda i: (0, i, 0))],
        out_specs=pl.BlockSpec((BN, CNT), lambda i: (i, 0)),
        out_shape=jax.ShapeDtypeStruct((n_nodes, CNT), jnp.float32),
    )(partials)

    return out[None]


# speedup vs baseline: 2.1997x; 1.0027x over previous
import functools

import jax
import jax.numpy as jnp
from jax import lax
from jax.experimental import pallas as pl
from jax.experimental.pallas import tpu as pltpu
from jax.experimental.pallas import tpu_sc as plsc

NS = 16
PAD = 32
CNT = 28

SC_CORES = 2
SC_SUBCORES = 16
NW = SC_CORES * SC_SUBCORES


def _node_body(x_ref, wtr_ref, a_ref, o_ref):
    xb = x_ref[...]
    wr = wtr_ref[..., 0] + wtr_ref[..., 1]
    a = a_ref[...]
    xs2 = jnp.dot(xb[:, :NS], wr, preferred_element_type=jnp.float32)
    y = (jnp.dot(xs2, a[:NS, :], preferred_element_type=jnp.float32)
         + jnp.dot(xb[:, NS:], a[NS:, :], preferred_element_type=jnp.float32))
    bn = y.shape[0]
    o_ref[...] = jnp.concatenate(
        [y, jnp.ones((bn, 1), jnp.float32), jnp.zeros((bn, PAD - CNT - 1), jnp.float32)],
        axis=1)


def _edge_body(ea_ref, es_ref, m1t_ref, b1_ref, m2t_ref, b2_ref, bsht_ref, o_ref):
    h = jnp.maximum(jnp.dot(m1t_ref[...], ea_ref[...],
                            preferred_element_type=jnp.float32) + b1_ref[...], 0.0)
    ew = jnp.dot(m2t_ref[...], h, preferred_element_type=jnp.float32) + b2_ref[...]
    shp = jnp.dot(bsht_ref[...], es_ref[...], preferred_element_type=jnp.float32)
    d = ew * shp
    be = d.shape[1]
    d32 = jnp.concatenate(
        [d, jnp.ones((1, be), jnp.float32), jnp.zeros((PAD - CNT - 1, be), jnp.float32)],
        axis=0)
    o_ref[...] = d32.reshape(PAD // 8, 8, be // 128, 128).transpose(2, 0, 1, 3)


def _combine_body(p_ref, o_ref):
    pb = p_ref[0] + p_ref[1]
    cnt = jnp.maximum(pb[:, CNT:CNT + 1], 1.0)
    o_ref[...] = pb[:, :CNT] / cnt


def _sc_scatter(n_nodes, n_edges):
    K = 128
    ncols = n_edges // K
    cols_base = ncols // NW
    cols_extra = ncols - cols_base * NW
    rows_per_sub = n_nodes // SC_SUBCORES
    ZR = 125
    nz = rows_per_sub // ZR

    mesh = plsc.VectorSubcoreMesh(core_axis_name="c", subcore_axis_name="s",
                                  num_cores=SC_CORES, num_subcores=SC_SUBCORES)

    scratch = [
        pltpu.VMEM((K,), jnp.int32),
        pltpu.VMEM((K,), jnp.int32),
        pltpu.VMEM((K, PAD), jnp.float32),
        pltpu.VMEM((PAD * K,), jnp.float32),
        pltpu.VMEM((ZR, PAD), jnp.float32),
        pltpu.VMEM_SHARED((n_nodes, PAD), jnp.float32),
        pltpu.SemaphoreType.DMA,
    ]

    @functools.partial(
        pl.kernel,
        out_type=jax.ShapeDtypeStruct((SC_CORES, n_nodes, PAD), jnp.float32),
        mesh=mesh,
        scratch_types=scratch,
        compiler_params=pltpu.CompilerParams(use_tc_tiling_on_sc=False,
                                             needs_layout_passes=False),
    )
    def run(src_hbm, dst_hbm, y_hbm, d_hbm, out_hbm, si, di, rows, dv, zbuf,
            acc, sem):
        c = lax.axis_index("c")
        s = lax.axis_index("s")
        wid = c * SC_SUBCORES + s
        zero16 = jnp.zeros((16,), jnp.float32)

        def zb(i, carry):
            zbuf[i, pl.ds(0, 16)] = zero16
            zbuf[i, pl.ds(16, 16)] = zero16
            return carry
        lax.fori_loop(0, ZR, zb, 0)

        def zc(k, carry):
            pltpu.sync_copy(zbuf, acc.at[pl.ds(s * rows_per_sub + k * ZR, ZR)])
            return carry
        lax.fori_loop(0, nz, zc, 0)
        plsc.subcore_barrier()

        col0 = wid * cols_base + jnp.minimum(wid, cols_extra)
        my_cols = cols_base + jnp.where(wid < cols_extra, 1, 0)

        idx0 = lax.iota(jnp.int32, 16) * K

        def chunk(ch, carry):
            col = col0 + ch
            base = col * K
            pltpu.sync_copy(src_hbm.at[pl.ds(base, K)], si)
            pltpu.sync_copy(dst_hbm.at[pl.ds(base, K)], di)
            pltpu.sync_copy(d_hbm.at[col], dv)
            pltpu.async_copy(y_hbm.at[si], rows, sem).wait()

            def mul(i, idx):
                v0 = plsc.load_gather(dv, [idx])
                v1 = plsc.load_gather(dv, [idx + 16 * K])
                rows[i, pl.ds(0, 16)] = rows[i, pl.ds(0, 16)] * v0
                rows[i, pl.ds(16, 16)] = rows[i, pl.ds(16, 16)] * v1
                return idx + 1
            lax.fori_loop(0, K, mul, idx0, unroll=4)
            pltpu.sync_copy(rows, acc.at[di], add=True)
            return carry
        lax.fori_loop(0, my_cols, chunk, 0)

        plsc.subcore_barrier()
        pltpu.sync_copy(acc.at[pl.ds(s * rows_per_sub, rows_per_sub)],
                        out_hbm.at[c, pl.ds(s * rows_per_sub, rows_per_sub)])

    return run


def kernel(x, edge_index, edge_attr, edge_sh, Wt_r, Wt_i, M1, b1, M2, b2, A, Bsh):
    n_nodes = x.shape[1]
    n_edges = edge_index.shape[1]
    src = edge_index[0]
    dst = edge_index[1]

    BN = 2000
    y_pad = pl.pallas_call(
        _node_body,
        grid=(n_nodes // BN,),
        in_specs=[
            pl.BlockSpec((BN, x.shape[2]), lambda i: (i, 0)),
            pl.BlockSpec(Wt_r.shape, lambda i: (0, 0, 0)),
            pl.BlockSpec(A.shape, lambda i: (0, 0)),
        ],
        out_specs=pl.BlockSpec((BN, PAD), lambda i: (i, 0)),
        out_shape=jax.ShapeDtypeStruct((n_nodes, PAD), jnp.float32),
    )(x[0], Wt_r, A)

    BE = 6400
    ea_t = edge_attr.T
    es_t = edge_sh.T
    d_pad_t = pl.pallas_call(
        _edge_body,
        grid=(n_edges // BE,),
        in_specs=[
            pl.BlockSpec((ea_t.shape[0], BE), lambda i: (0, i)),
            pl.BlockSpec((es_t.shape[0], BE), lambda i: (0, i)),
            pl.BlockSpec(M1.shape, lambda i: (0, 0)),
            pl.BlockSpec((b1.shape[0], 1), lambda i: (0, 0)),
            pl.BlockSpec((M2.shape[1], M2.shape[0]), lambda i: (0, 0)),
            pl.BlockSpec((b2.shape[0], 1), lambda i: (0, 0)),
            pl.BlockSpec((Bsh.shape[1], Bsh.shape[0]), lambda i: (0, 0)),
        ],
        out_specs=pl.BlockSpec((BE // 128, PAD // 8, 8, 128), lambda i: (i, 0, 0, 0)),
        out_shape=jax.ShapeDtypeStruct((n_edges // 128, PAD // 8, 8, 128),
                                       jnp.float32),
    )(ea_t, es_t, M1.T, b1.reshape(-1, 1), M2.T, b2.reshape(-1, 1), Bsh.T)
    d_lin = d_pad_t.reshape(n_edges // 128, PAD * 128)

    partials = _sc_scatter(n_nodes, n_edges)(src, dst, y_pad, d_lin)

    out = pl.pallas_call(
        _combine_body,
        grid=(n_nodes // BN,),
        in_specs=[pl.BlockSpec((SC_CORES, BN, PAD), lambda i: (0, i, 0))],
        out_specs=pl.BlockSpec((BN, CNT), lambda i: (i, 0)),
        out_shape=jax.ShapeDtypeStruct((n_nodes, CNT), jnp.float32),
    )(partials)

    return out[None]


# speedup vs baseline: 2.9758x; 1.3528x over previous
import functools

import jax
import jax.numpy as jnp
from jax import lax
from jax.experimental import pallas as pl
from jax.experimental.pallas import tpu as pltpu
from jax.experimental.pallas import tpu_sc as plsc

NS = 16
PAD = 32
CNT = 28

SC_CORES = 2
SC_SUBCORES = 16
NW = SC_CORES * SC_SUBCORES


def _node_body(x_ref, wtr_ref, a_ref, o_ref):
    xb = x_ref[...]
    wr = wtr_ref[..., 0] + wtr_ref[..., 1]
    a = a_ref[...]
    xs2 = jnp.dot(xb[:, :NS], wr, preferred_element_type=jnp.float32)
    y = (jnp.dot(xs2, a[:NS, :], preferred_element_type=jnp.float32)
         + jnp.dot(xb[:, NS:], a[NS:, :], preferred_element_type=jnp.float32))
    bn = y.shape[0]
    o_ref[...] = jnp.concatenate(
        [y, jnp.ones((bn, 1), jnp.float32), jnp.zeros((bn, PAD - CNT - 1), jnp.float32)],
        axis=1)


def _edge_body(ea_ref, es_ref, m1t_ref, b1_ref, m2t_ref, b2_ref, bsht_ref, o_ref):
    h = jnp.maximum(jnp.dot(m1t_ref[...], ea_ref[...],
                            preferred_element_type=jnp.float32) + b1_ref[...], 0.0)
    ew = jnp.dot(m2t_ref[...], h, preferred_element_type=jnp.float32) + b2_ref[...]
    shp = jnp.dot(bsht_ref[...], es_ref[...], preferred_element_type=jnp.float32)
    d = ew * shp
    be = d.shape[1]
    d32 = jnp.concatenate(
        [d, jnp.ones((1, be), jnp.float32), jnp.zeros((PAD - CNT - 1, be), jnp.float32)],
        axis=0)
    o_ref[...] = d32.reshape(PAD // 8, 8, be // 128, 128).transpose(2, 0, 1, 3)


def _combine_body(p_ref, o_ref):
    pb = p_ref[0] + p_ref[1]
    cnt = jnp.maximum(pb[:, CNT:CNT + 1], 1.0)
    o_ref[...] = pb[:, :CNT] / cnt


def _sc_scatter(n_nodes, n_edges):
    K = 128
    ncols = n_edges // K
    cols_base = ncols // NW
    cols_extra = ncols - cols_base * NW
    rows_per_sub = n_nodes // SC_SUBCORES
    ZR = 125
    nz = rows_per_sub // ZR

    mesh = plsc.VectorSubcoreMesh(core_axis_name="c", subcore_axis_name="s",
                                  num_cores=SC_CORES, num_subcores=SC_SUBCORES)

    scratch = [
        pltpu.VMEM((K,), jnp.int32),
        pltpu.VMEM((K,), jnp.int32),
        pltpu.VMEM((K, PAD), jnp.float32),
        pltpu.VMEM((PAD * K,), jnp.float32),
        pltpu.VMEM((ZR, PAD), jnp.float32),
        pltpu.VMEM_SHARED((n_nodes, PAD), jnp.float32),
        pltpu.SemaphoreType.DMA,
    ]

    @functools.partial(
        pl.kernel,
        out_type=jax.ShapeDtypeStruct((SC_CORES, n_nodes, PAD), jnp.float32),
        mesh=mesh,
        scratch_types=scratch,
        compiler_params=pltpu.CompilerParams(use_tc_tiling_on_sc=False,
                                             needs_layout_passes=False),
    )
    def run(src_hbm, dst_hbm, y_hbm, d_hbm, out_hbm, si, di, rows, dv, zbuf,
            acc, sem):
        c = lax.axis_index("c")
        s = lax.axis_index("s")
        wid = c * SC_SUBCORES + s
        zero16 = jnp.zeros((16,), jnp.float32)

        def zb(i, carry):
            zbuf[i, pl.ds(0, 16)] = zero16
            zbuf[i, pl.ds(16, 16)] = zero16
            return carry
        lax.fori_loop(0, ZR, zb, 0)

        def zc(k, carry):
            pltpu.sync_copy(zbuf, acc.at[pl.ds(s * rows_per_sub + k * ZR, ZR)])
            return carry
        lax.fori_loop(0, nz, zc, 0)
        plsc.subcore_barrier()

        col0 = wid * cols_base + jnp.minimum(wid, cols_extra)
        my_cols = cols_base + jnp.where(wid < cols_extra, 1, 0)

        iota16 = lax.iota(jnp.int32, 16)
        f_lo = iota16
        f_hi = iota16 + 16
        d_lo = iota16 * K
        d_hi = d_lo + 16 * K
        ek = [(iota16 + k) & 15 for k in range(16)]

        def chunk(ch, carry):
            col = col0 + ch
            base = col * K
            pltpu.sync_copy(src_hbm.at[pl.ds(base, K)], si)
            pltpu.sync_copy(dst_hbm.at[pl.ds(base, K)], di)
            pltpu.sync_copy(d_hbm.at[col], dv)
            pltpu.async_copy(y_hbm.at[si], rows, sem).wait()

            def mulgrp(j, carry2):
                e0 = jnp.full((16,), j * 16, jnp.int32)
                for k in range(16):
                    idx_e = e0 + ek[k]
                    vd0 = plsc.load_gather(dv, [d_lo + idx_e])
                    vd1 = plsc.load_gather(dv, [d_hi + idx_e])
                    vy0 = plsc.load_gather(rows, [idx_e, f_lo])
                    vy1 = plsc.load_gather(rows, [idx_e, f_hi])
                    plsc.store_scatter(rows, [idx_e, f_lo], vy0 * vd0)
                    plsc.store_scatter(rows, [idx_e, f_hi], vy1 * vd1)
                return carry2
            lax.fori_loop(0, K // 16, mulgrp, 0)
            pltpu.sync_copy(rows, acc.at[di], add=True)
            return carry
        lax.fori_loop(0, my_cols, chunk, 0)

        plsc.subcore_barrier()
        pltpu.sync_copy(acc.at[pl.ds(s * rows_per_sub, rows_per_sub)],
                        out_hbm.at[c, pl.ds(s * rows_per_sub, rows_per_sub)])

    return run


def kernel(x, edge_index, edge_attr, edge_sh, Wt_r, Wt_i, M1, b1, M2, b2, A, Bsh):
    n_nodes = x.shape[1]
    n_edges = edge_index.shape[1]
    src = edge_index[0]
    dst = edge_index[1]

    BN = 2000
    y_pad = pl.pallas_call(
        _node_body,
        grid=(n_nodes // BN,),
        in_specs=[
            pl.BlockSpec((BN, x.shape[2]), lambda i: (i, 0)),
            pl.BlockSpec(Wt_r.shape, lambda i: (0, 0, 0)),
            pl.BlockSpec(A.shape, lambda i: (0, 0)),
        ],
        out_specs=pl.BlockSpec((BN, PAD), lambda i: (i, 0)),
        out_shape=jax.ShapeDtypeStruct((n_nodes, PAD), jnp.float32),
    )(x[0], Wt_r, A)

    BE = 6400
    ea_t = edge_attr.T
    es_t = edge_sh.T
    d_pad_t = pl.pallas_call(
        _edge_body,
        grid=(n_edges // BE,),
        in_specs=[
            pl.BlockSpec((ea_t.shape[0], BE), lambda i: (0, i)),
            pl.BlockSpec((es_t.shape[0], BE), lambda i: (0, i)),
            pl.BlockSpec(M1.shape, lambda i: (0, 0)),
            pl.BlockSpec((b1.shape[0], 1), lambda i: (0, 0)),
            pl.BlockSpec((M2.shape[1], M2.shape[0]), lambda i: (0, 0)),
            pl.BlockSpec((b2.shape[0], 1), lambda i: (0, 0)),
            pl.BlockSpec((Bsh.shape[1], Bsh.shape[0]), lambda i: (0, 0)),
        ],
        out_specs=pl.BlockSpec((BE // 128, PAD // 8, 8, 128), lambda i: (i, 0, 0, 0)),
        out_shape=jax.ShapeDtypeStruct((n_edges // 128, PAD // 8, 8, 128),
                                       jnp.float32),
    )(ea_t, es_t, M1.T, b1.reshape(-1, 1), M2.T, b2.reshape(-1, 1), Bsh.T)
    d_lin = d_pad_t.reshape(n_edges // 128, PAD * 128)

    partials = _sc_scatter(n_nodes, n_edges)(src, dst, y_pad, d_lin)

    out = pl.pallas_call(
        _combine_body,
        grid=(n_nodes // BN,),
        in_specs=[pl.BlockSpec((SC_CORES, BN, PAD), lambda i: (0, i, 0))],
        out_specs=pl.BlockSpec((BN, CNT), lambda i: (i, 0)),
        out_shape=jax.ShapeDtypeStruct((n_nodes, CNT), jnp.float32),
    )(partials)

    return out[None]


# speedup vs baseline: 4.4723x; 1.5029x over previous
import functools

import jax
import jax.numpy as jnp
from jax import lax
from jax.experimental import pallas as pl
from jax.experimental.pallas import tpu as pltpu
from jax.experimental.pallas import tpu_sc as plsc

NS = 16
PAD = 32
CNT = 28

SC_CORES = 2
SC_SUBCORES = 16
NW = SC_CORES * SC_SUBCORES


def _node_body(x_ref, wtr_ref, a_ref, o_ref):
    xb = x_ref[...]
    wr = wtr_ref[..., 0] + wtr_ref[..., 1]
    a = a_ref[...]
    xs2 = jnp.dot(xb[:, :NS], wr, preferred_element_type=jnp.float32)
    y = (jnp.dot(xs2, a[:NS, :], preferred_element_type=jnp.float32)
         + jnp.dot(xb[:, NS:], a[NS:, :], preferred_element_type=jnp.float32))
    bn = y.shape[0]
    o_ref[...] = jnp.concatenate(
        [y, jnp.ones((bn, 1), jnp.float32), jnp.zeros((bn, PAD - CNT - 1), jnp.float32)],
        axis=1)


def _edge_body(ea_ref, es_ref, m1t_ref, b1_ref, m2t_ref, b2_ref, bsht_ref, o_ref):
    h = jnp.maximum(jnp.dot(m1t_ref[...], ea_ref[...],
                            preferred_element_type=jnp.float32) + b1_ref[...], 0.0)
    ew = jnp.dot(m2t_ref[...], h, preferred_element_type=jnp.float32) + b2_ref[...]
    shp = jnp.dot(bsht_ref[...], es_ref[...], preferred_element_type=jnp.float32)
    d = ew * shp
    be = d.shape[1]
    d32 = jnp.concatenate(
        [d, jnp.ones((1, be), jnp.float32), jnp.zeros((PAD - CNT - 1, be), jnp.float32)],
        axis=0)
    o_ref[...] = d32.reshape(PAD // 8, 8, be // 128, 128).transpose(2, 0, 1, 3)


def _combine_body(p_ref, o_ref):
    pb = p_ref[0] + p_ref[1]
    cnt = jnp.maximum(pb[:, CNT:CNT + 1], 1.0)
    o_ref[...] = pb[:, :CNT] / cnt


def _sc_scatter(n_nodes, n_edges):
    K = 128
    ncols = n_edges // K
    cols_base = ncols // NW
    cols_extra = ncols - cols_base * NW
    rows_per_sub = n_nodes // SC_SUBCORES
    ZR = 125
    nz = rows_per_sub // ZR

    mesh = plsc.VectorSubcoreMesh(core_axis_name="c", subcore_axis_name="s",
                                  num_cores=SC_CORES, num_subcores=SC_SUBCORES)

    buf = lambda: [
        pltpu.VMEM((K,), jnp.int32),
        pltpu.VMEM((K,), jnp.int32),
        pltpu.VMEM((PAD * K,), jnp.float32),
        pltpu.VMEM((K, PAD), jnp.float32),
        pltpu.SemaphoreType.DMA,
        pltpu.SemaphoreType.DMA,
        pltpu.SemaphoreType.DMA,
    ]
    scratch = buf() + buf() + [
        pltpu.VMEM((ZR, PAD), jnp.float32),
        pltpu.VMEM_SHARED((n_nodes, PAD), jnp.float32),
    ]

    @functools.partial(
        pl.kernel,
        out_type=jax.ShapeDtypeStruct((SC_CORES, n_nodes, PAD), jnp.float32),
        mesh=mesh,
        scratch_types=scratch,
        compiler_params=pltpu.CompilerParams(use_tc_tiling_on_sc=False,
                                             needs_layout_passes=False),
    )
    def run(src_hbm, dst_hbm, y_hbm, d_hbm, out_hbm,
            si0, di0, dv0, rows0, semio0, semg0, sems0,
            si1, di1, dv1, rows1, semio1, semg1, sems1,
            zbuf, acc):
        c = lax.axis_index("c")
        s = lax.axis_index("s")
        wid = c * SC_SUBCORES + s
        zero16 = jnp.zeros((16,), jnp.float32)

        def zb(i, carry):
            zbuf[i, pl.ds(0, 16)] = zero16
            zbuf[i, pl.ds(16, 16)] = zero16
            return carry
        lax.fori_loop(0, ZR, zb, 0)

        def zc(k, carry):
            pltpu.sync_copy(zbuf, acc.at[pl.ds(s * rows_per_sub + k * ZR, ZR)])
            return carry
        lax.fori_loop(0, nz, zc, 0)
        plsc.subcore_barrier()

        col0 = wid * cols_base + jnp.minimum(wid, cols_extra)
        my_cols = cols_base + jnp.where(wid < cols_extra, 1, 0)

        iota16 = lax.iota(jnp.int32, 16)
        f_lo = iota16
        f_hi = iota16 + 16
        d_lo = iota16 * K
        d_hi = d_lo + 16 * K
        ek = [(iota16 + k) & 15 for k in range(16)]

        def do_mul(rows_, dv_):
            def mulgrp(j, carry2):
                e0 = jnp.full((16,), j * 16, jnp.int32)
                for k in range(16):
                    idx_e = e0 + ek[k]
                    vd0 = plsc.load_gather(dv_, [d_lo + idx_e])
                    vd1 = plsc.load_gather(dv_, [d_hi + idx_e])
                    vy0 = plsc.load_gather(rows_, [idx_e, f_lo])
                    vy1 = plsc.load_gather(rows_, [idx_e, f_hi])
                    plsc.store_scatter(rows_, [idx_e, f_lo], vy0 * vd0)
                    plsc.store_scatter(rows_, [idx_e, f_hi], vy1 * vd1)
                return carry2
            lax.fori_loop(0, K // 16, mulgrp, 0)

        def issue_io(col, si_, di_, dv_, sem_):
            base = col * K
            pltpu.async_copy(src_hbm.at[pl.ds(base, K)], si_, sem_)
            pltpu.async_copy(dst_hbm.at[pl.ds(base, K)], di_, sem_)
            pltpu.async_copy(d_hbm.at[col], dv_, sem_)

        def wait_io(si_, di_, dv_, sem_):
            pltpu.make_async_copy(src_hbm.at[pl.ds(0, K)], si_, sem_).wait()
            pltpu.make_async_copy(dst_hbm.at[pl.ds(0, K)], di_, sem_).wait()
            pltpu.make_async_copy(d_hbm.at[0], dv_, sem_).wait()

        npairs = my_cols // 2
        odd = my_cols - npairs * 2

        issue_io(col0, si0, di0, dv0, semio0)
        issue_io(col0 + 1, si1, di1, dv1, semio1)

        def pair(g, carry):
            c0 = col0 + 2 * g
            wait_io(si0, di0, dv0, semio0)
            g0 = pltpu.async_copy(y_hbm.at[si0], rows0, semg0)
            wait_io(si1, di1, dv1, semio1)
            g1 = pltpu.async_copy(y_hbm.at[si1], rows1, semg1)
            g0.wait()
            do_mul(rows0, dv0)
            pltpu.async_copy(rows0, acc.at[di0], sems0, add=True)
            g1.wait()
            do_mul(rows1, dv1)
            pltpu.async_copy(rows1, acc.at[di1], sems1, add=True)
            pltpu.make_async_copy(rows0, acc.at[di0], sems0).wait()

            @pl.when(2 * g + 2 < my_cols)
            def _():
                issue_io(c0 + 2, si0, di0, dv0, semio0)
            pltpu.make_async_copy(rows1, acc.at[di1], sems1).wait()

            @pl.when(2 * g + 3 < my_cols)
            def _():
                issue_io(c0 + 3, si1, di1, dv1, semio1)
            return carry
        lax.fori_loop(0, npairs, pair, 0)

        @pl.when(odd == 1)
        def _():
            wait_io(si0, di0, dv0, semio0)
            pltpu.async_copy(y_hbm.at[si0], rows0, semg0).wait()
            do_mul(rows0, dv0)
            pltpu.sync_copy(rows0, acc.at[di0], add=True)

        plsc.subcore_barrier()
        pltpu.sync_copy(acc.at[pl.ds(s * rows_per_sub, rows_per_sub)],
                        out_hbm.at[c, pl.ds(s * rows_per_sub, rows_per_sub)])

    return run


def kernel(x, edge_index, edge_attr, edge_sh, Wt_r, Wt_i, M1, b1, M2, b2, A, Bsh):
    n_nodes = x.shape[1]
    n_edges = edge_index.shape[1]
    src = edge_index[0]
    dst = edge_index[1]

    BN = 2000
    y_pad = pl.pallas_call(
        _node_body,
        grid=(n_nodes // BN,),
        in_specs=[
            pl.BlockSpec((BN, x.shape[2]), lambda i: (i, 0)),
            pl.BlockSpec(Wt_r.shape, lambda i: (0, 0, 0)),
            pl.BlockSpec(A.shape, lambda i: (0, 0)),
        ],
        out_specs=pl.BlockSpec((BN, PAD), lambda i: (i, 0)),
        out_shape=jax.ShapeDtypeStruct((n_nodes, PAD), jnp.float32),
    )(x[0], Wt_r, A)

    BE = 6400
    ea_t = edge_attr.T
    es_t = edge_sh.T
    d_pad_t = pl.pallas_call(
        _edge_body,
        grid=(n_edges // BE,),
        in_specs=[
            pl.BlockSpec((ea_t.shape[0], BE), lambda i: (0, i)),
            pl.BlockSpec((es_t.shape[0], BE), lambda i: (0, i)),
            pl.BlockSpec(M1.shape, lambda i: (0, 0)),
            pl.BlockSpec((b1.shape[0], 1), lambda i: (0, 0)),
            pl.BlockSpec((M2.shape[1], M2.shape[0]), lambda i: (0, 0)),
            pl.BlockSpec((b2.shape[0], 1), lambda i: (0, 0)),
            pl.BlockSpec((Bsh.shape[1], Bsh.shape[0]), lambda i: (0, 0)),
        ],
        out_specs=pl.BlockSpec((BE // 128, PAD // 8, 8, 128), lambda i: (i, 0, 0, 0)),
        out_shape=jax.ShapeDtypeStruct((n_edges // 128, PAD // 8, 8, 128),
                                       jnp.float32),
    )(ea_t, es_t, M1.T, b1.reshape(-1, 1), M2.T, b2.reshape(-1, 1), Bsh.T)
    d_lin = d_pad_t.reshape(n_edges // 128, PAD * 128)

    partials = _sc_scatter(n_nodes, n_edges)(src, dst, y_pad, d_lin)

    out = pl.pallas_call(
        _combine_body,
        grid=(n_nodes // BN,),
        in_specs=[pl.BlockSpec((SC_CORES, BN, PAD), lambda i: (0, i, 0))],
        out_specs=pl.BlockSpec((BN, CNT), lambda i: (i, 0)),
        out_shape=jax.ShapeDtypeStruct((n_nodes, CNT), jnp.float32),
    )(partials)

    return out[None]


# speedup vs baseline: 4.6565x; 1.0412x over previous
import functools

import jax
import jax.numpy as jnp
from jax import lax
from jax.experimental import pallas as pl
from jax.experimental.pallas import tpu as pltpu
from jax.experimental.pallas import tpu_sc as plsc

NS = 16
PAD = 32
CNT = 28

SC_CORES = 2
SC_SUBCORES = 16
NW = SC_CORES * SC_SUBCORES


def _node_body(xt_ref, wtr_ref, a_ref, o_ref):
    xt = xt_ref[...]
    wr = wtr_ref[..., 0] + wtr_ref[..., 1]
    a = a_ref[...]
    cdim0 = (((0,), (0,)), ((), ()))
    xs2_t = lax.dot_general(wr, xt[:NS], cdim0,
                            preferred_element_type=jnp.float32)
    x2_t = jnp.concatenate([xs2_t, xt[NS:]], axis=0)
    y = lax.dot_general(x2_t, a, cdim0,
                        preferred_element_type=jnp.float32)
    bn = y.shape[0]
    o_ref[...] = jnp.concatenate(
        [y, jnp.ones((bn, 1), jnp.float32), jnp.zeros((bn, PAD - CNT - 1), jnp.float32)],
        axis=1)


def _edge_body(ea_ref, es_ref, m1t_ref, b1_ref, m2t_ref, b2_ref, bsht_ref, o_ref):
    h = jnp.maximum(jnp.dot(m1t_ref[...], ea_ref[...],
                            preferred_element_type=jnp.float32) + b1_ref[...], 0.0)
    ew = jnp.dot(m2t_ref[...], h, preferred_element_type=jnp.float32) + b2_ref[...]
    shp = jnp.dot(bsht_ref[...], es_ref[...], preferred_element_type=jnp.float32)
    d = ew * shp
    be = d.shape[1]
    d32 = jnp.concatenate(
        [d, jnp.ones((1, be), jnp.float32), jnp.zeros((PAD - CNT - 1, be), jnp.float32)],
        axis=0)
    o_ref[...] = d32.reshape(PAD // 8, 8, be // 128, 128).transpose(2, 0, 1, 3)


def _combine_body(p_ref, o_ref):
    pb = p_ref[0] + p_ref[1]
    cnt = jnp.maximum(pb[:, CNT:CNT + 1], 1.0)
    o_ref[...] = pb[:, :CNT] / cnt


def _sc_scatter(n_nodes, n_edges):
    K = 128
    ncols = n_edges // K
    cols_base = ncols // NW
    cols_extra = ncols - cols_base * NW
    rows_per_sub = n_nodes // SC_SUBCORES
    ZR = 125
    nz = rows_per_sub // ZR

    mesh = plsc.VectorSubcoreMesh(core_axis_name="c", subcore_axis_name="s",
                                  num_cores=SC_CORES, num_subcores=SC_SUBCORES)

    buf = lambda: [
        pltpu.VMEM((K,), jnp.int32),
        pltpu.VMEM((K,), jnp.int32),
        pltpu.VMEM((PAD * K,), jnp.float32),
        pltpu.VMEM((K, PAD), jnp.float32),
        pltpu.SemaphoreType.DMA,
        pltpu.SemaphoreType.DMA,
        pltpu.SemaphoreType.DMA,
    ]
    scratch = buf() + buf() + [
        pltpu.VMEM((ZR, PAD), jnp.float32),
        pltpu.VMEM_SHARED((n_nodes, PAD), jnp.float32),
    ]

    @functools.partial(
        pl.kernel,
        out_type=jax.ShapeDtypeStruct((SC_CORES, n_nodes, PAD), jnp.float32),
        mesh=mesh,
        scratch_types=scratch,
        compiler_params=pltpu.CompilerParams(use_tc_tiling_on_sc=False,
                                             needs_layout_passes=False),
    )
    def run(src_hbm, dst_hbm, y_hbm, d_hbm, out_hbm,
            si0, di0, dv0, rows0, semio0, semg0, sems0,
            si1, di1, dv1, rows1, semio1, semg1, sems1,
            zbuf, acc):
        c = lax.axis_index("c")
        s = lax.axis_index("s")
        wid = c * SC_SUBCORES + s
        zero16 = jnp.zeros((16,), jnp.float32)

        def zb(i, carry):
            zbuf[i, pl.ds(0, 16)] = zero16
            zbuf[i, pl.ds(16, 16)] = zero16
            return carry
        lax.fori_loop(0, ZR, zb, 0)

        def zc(k, carry):
            pltpu.sync_copy(zbuf, acc.at[pl.ds(s * rows_per_sub + k * ZR, ZR)])
            return carry
        lax.fori_loop(0, nz, zc, 0)
        plsc.subcore_barrier()

        col0 = wid * cols_base + jnp.minimum(wid, cols_extra)
        my_cols = cols_base + jnp.where(wid < cols_extra, 1, 0)

        iota16 = lax.iota(jnp.int32, 16)
        f_lo = iota16
        f_hi = iota16 + 16
        d_lo = iota16 * K
        d_hi = d_lo + 16 * K
        ek = [(iota16 + k) & 15 for k in range(16)]

        def do_mul(rows_, dv_):
            def mulgrp(j, carry2):
                e0 = jnp.full((16,), j * 16, jnp.int32)
                for k in range(16):
                    idx_e = e0 + ek[k]
                    vd0 = plsc.load_gather(dv_, [d_lo + idx_e])
                    vd1 = plsc.load_gather(dv_, [d_hi + idx_e])
                    vy0 = plsc.load_gather(rows_, [idx_e, f_lo])
                    vy1 = plsc.load_gather(rows_, [idx_e, f_hi])
                    plsc.store_scatter(rows_, [idx_e, f_lo], vy0 * vd0)
                    plsc.store_scatter(rows_, [idx_e, f_hi], vy1 * vd1)
                return carry2
            lax.fori_loop(0, K // 16, mulgrp, 0)

        def issue_io(col, si_, di_, dv_, sem_):
            base = col * K
            pltpu.async_copy(src_hbm.at[pl.ds(base, K)], si_, sem_)
            pltpu.async_copy(dst_hbm.at[pl.ds(base, K)], di_, sem_)
            pltpu.async_copy(d_hbm.at[col], dv_, sem_)

        def wait_io(si_, di_, dv_, sem_):
            pltpu.make_async_copy(src_hbm.at[pl.ds(0, K)], si_, sem_).wait()
            pltpu.make_async_copy(dst_hbm.at[pl.ds(0, K)], di_, sem_).wait()
            pltpu.make_async_copy(d_hbm.at[0], dv_, sem_).wait()

        npairs = my_cols // 2
        odd = my_cols - npairs * 2

        issue_io(col0, si0, di0, dv0, semio0)
        issue_io(col0 + 1, si1, di1, dv1, semio1)

        def pair(g, carry):
            c0 = col0 + 2 * g
            wait_io(si0, di0, dv0, semio0)
            g0 = pltpu.async_copy(y_hbm.at[si0], rows0, semg0)
            wait_io(si1, di1, dv1, semio1)
            g1 = pltpu.async_copy(y_hbm.at[si1], rows1, semg1)
            g0.wait()
            do_mul(rows0, dv0)
            pltpu.async_copy(rows0, acc.at[di0], sems0, add=True)
            g1.wait()
            do_mul(rows1, dv1)
            pltpu.async_copy(rows1, acc.at[di1], sems1, add=True)
            pltpu.make_async_copy(rows0, acc.at[di0], sems0).wait()

            @pl.when(2 * g + 2 < my_cols)
            def _():
                issue_io(c0 + 2, si0, di0, dv0, semio0)
            pltpu.make_async_copy(rows1, acc.at[di1], sems1).wait()

            @pl.when(2 * g + 3 < my_cols)
            def _():
                issue_io(c0 + 3, si1, di1, dv1, semio1)
            return carry
        lax.fori_loop(0, npairs, pair, 0)

        @pl.when(odd == 1)
        def _():
            wait_io(si0, di0, dv0, semio0)
            pltpu.async_copy(y_hbm.at[si0], rows0, semg0).wait()
            do_mul(rows0, dv0)
            pltpu.sync_copy(rows0, acc.at[di0], add=True)

        plsc.subcore_barrier()
        pltpu.sync_copy(acc.at[pl.ds(s * rows_per_sub, rows_per_sub)],
                        out_hbm.at[c, pl.ds(s * rows_per_sub, rows_per_sub)])

    return run


def kernel(x, edge_index, edge_attr, edge_sh, Wt_r, Wt_i, M1, b1, M2, b2, A, Bsh):
    n_nodes = x.shape[1]
    n_edges = edge_index.shape[1]
    src = edge_index[0]
    dst = edge_index[1]

    BN = 2000
    x_t = x[0].T
    y_pad = pl.pallas_call(
        _node_body,
        grid=(1,),
        in_specs=[
            pl.BlockSpec(x_t.shape, lambda i: (0, 0)),
            pl.BlockSpec(Wt_r.shape, lambda i: (0, 0, 0)),
            pl.BlockSpec(A.shape, lambda i: (0, 0)),
        ],
        out_specs=pl.BlockSpec((n_nodes, PAD), lambda i: (0, 0)),
        out_shape=jax.ShapeDtypeStruct((n_nodes, PAD), jnp.float32),
    )(x_t, Wt_r, A)

    BE = 6400
    ea_t = edge_attr.T
    es_t = edge_sh.T
    d_pad_t = pl.pallas_call(
        _edge_body,
        grid=(n_edges // BE,),
        in_specs=[
            pl.BlockSpec((ea_t.shape[0], BE), lambda i: (0, i)),
            pl.BlockSpec((es_t.shape[0], BE), lambda i: (0, i)),
            pl.BlockSpec(M1.shape, lambda i: (0, 0)),
            pl.BlockSpec((b1.shape[0], 1), lambda i: (0, 0)),
            pl.BlockSpec((M2.shape[1], M2.shape[0]), lambda i: (0, 0)),
            pl.BlockSpec((b2.shape[0], 1), lambda i: (0, 0)),
            pl.BlockSpec((Bsh.shape[1], Bsh.shape[0]), lambda i: (0, 0)),
        ],
        out_specs=pl.BlockSpec((BE // 128, PAD // 8, 8, 128), lambda i: (i, 0, 0, 0)),
        out_shape=jax.ShapeDtypeStruct((n_edges // 128, PAD // 8, 8, 128),
                                       jnp.float32),
    )(ea_t, es_t, M1.T, b1.reshape(-1, 1), M2.T, b2.reshape(-1, 1), Bsh.T)
    d_lin = d_pad_t.reshape(n_edges // 128, PAD * 128)

    partials = _sc_scatter(n_nodes, n_edges)(src, dst, y_pad, d_lin)

    out = pl.pallas_call(
        _combine_body,
        grid=(n_nodes // BN,),
        in_specs=[pl.BlockSpec((SC_CORES, BN, PAD), lambda i: (0, i, 0))],
        out_specs=pl.BlockSpec((BN, CNT), lambda i: (i, 0)),
        out_shape=jax.ShapeDtypeStruct((n_nodes, CNT), jnp.float32),
    )(partials)

    return out[None]
